# Initial kernel scaffold; baseline (speedup 1.0000x reference)
import jax
import jax.numpy as jnp
from jax.experimental import pallas as pl


def kernel(ego_embeddings, edge_index, edge_vals, ln0_gamma, ln0_beta, ln1_gamma, ln1_beta):
    raise NotImplementedError("write your pallas kernel here")


# speedup vs baseline: 1.0108x; 1.0108x over previous
import functools

import jax
import jax.numpy as jnp
from jax.experimental import pallas as pl

N_NODES = 10000
D = 128
LEAKY = 0.2


def _post_body(h_ref, res_ref, g_ref, b_ref, o_ref, *, leaky):
    h = h_ref[...]
    if leaky:
        h = jnp.where(h > 0, h, LEAKY * h)
    mu = jnp.mean(h, axis=-1, keepdims=True)
    var = jnp.mean((h - mu) ** 2, axis=-1, keepdims=True)
    ln = (h - mu) * jax.lax.rsqrt(var + 1e-5) * g_ref[...] + b_ref[...]
    o_ref[...] = ln + res_ref[...]


def _post(h, res, gamma, beta, leaky):
    blk = 400
    grid = (N_NODES // blk,)
    return pl.pallas_call(
        functools.partial(_post_body, leaky=leaky),
        grid=grid,
        in_specs=[
            pl.BlockSpec((blk, D), lambda i: (i, 0)),
            pl.BlockSpec((blk, D), lambda i: (i, 0)),
            pl.BlockSpec((D,), lambda i: (0,)),
            pl.BlockSpec((D,), lambda i: (0,)),
        ],
        out_specs=pl.BlockSpec((blk, D), lambda i: (i, 0)),
        out_shape=jax.ShapeDtypeStruct((N_NODES, D), jnp.float32),
    )(h, res, gamma, beta)


def kernel(ego_embeddings, edge_index, edge_vals, ln0_gamma, ln0_beta, ln1_gamma, ln1_beta):
    rows = edge_index[0]
    cols = edge_index[1]
    res = ego_embeddings
    e = ego_embeddings
    tmp = jax.ops.segment_sum(edge_vals[:, None] * e[rows], cols, num_segments=N_NODES)
    h = jax.ops.segment_sum(edge_vals[:, None] * tmp[cols], rows, num_segments=N_NODES)
    e = _post(h, res, ln0_gamma, ln0_beta, leaky=True)
    tmp = jax.ops.segment_sum(edge_vals[:, None] * e[rows], cols, num_segments=N_NODES)
    h = jax.ops.segment_sum(edge_vals[:, None] * tmp[cols], rows, num_segments=N_NODES)
    e = _post(h, res, ln1_gamma, ln1_beta, leaky=False)
    return (e[:5000], e[5000:])


# speedup vs baseline: 2.8329x; 2.8026x over previous
import functools

import jax
import jax.numpy as jnp
from jax import lax
from jax.experimental import pallas as pl
from jax.experimental.pallas import tpu as pltpu
from jax.experimental.pallas import tpu_sc as plsc

N_NODES = 10000
N_PAD = 10240
N_USERS = 5000
N_EDGES = 320000
D = 128
LEAKY = 0.2

NC = 2
NS = 16
NW = NC * NS
EPW = N_EDGES // NW
SUB = 128
EPW_PAD = 10240
NSUB = EPW_PAD // SUB
E_PAD = NW * EPW_PAD
RPT = N_PAD // NS

_mesh = plsc.VectorSubcoreMesh(core_axis_name="c", subcore_axis_name="s")


@functools.partial(
    pl.kernel,
    out_type=jax.ShapeDtypeStruct((NC, N_PAD, D), jnp.float32),
    mesh=_mesh,
    scratch_types=[
        pltpu.VMEM((NSUB, SUB), jnp.int32),
        pltpu.VMEM((NSUB, SUB), jnp.int32),
        pltpu.VMEM((NSUB, SUB), jnp.float32),
        pltpu.VMEM((SUB, D), jnp.float32),
        pltpu.VMEM_SHARED((N_PAD, D), jnp.float32),
        pltpu.SemaphoreType.DMA,
    ],
)
def _spmm(table, gidx, sidx, vals, out, gidx_v, sidx_v, vals_v, rows_v, acc, sem):
    cid = lax.axis_index("c")
    sid = lax.axis_index("s")
    wid = cid * NS + sid

    pltpu.sync_copy(gidx.at[wid], gidx_v)
    pltpu.sync_copy(sidx.at[wid], sidx_v)
    pltpu.sync_copy(vals.at[wid], vals_v)

    def _zrow(e, carry):
        for j in range(D // 16):
            rows_v[e, pl.ds(j * 16, 16)] = jnp.zeros((16,), jnp.float32)
        return carry

    lax.fori_loop(0, SUB, _zrow, 0)
    row0 = sid * RPT
    for k in range(RPT // SUB):
        pltpu.sync_copy(rows_v, acc.at[pl.ds(row0 + k * SUB, SUB)])
    plsc.subcore_barrier()

    def _sub(s, c1):
        pltpu.async_copy(table.at[gidx_v.at[s]], rows_v, sem).wait()

        def _scale(grp, c2):
            e0 = grp * 16
            val16 = vals_v[s, pl.ds(e0, 16)]
            for l in range(16):
                v = val16[l]
                for j in range(D // 16):
                    rows_v[e0 + l, pl.ds(j * 16, 16)] = (
                        rows_v[e0 + l, pl.ds(j * 16, 16)] * v)
            return c2

        lax.fori_loop(0, SUB // 16, _scale, 0)
        pltpu.sync_copy(rows_v, acc.at[sidx_v.at[s]], add=True)
        return c1

    lax.fori_loop(0, NSUB, _sub, 0)
    plsc.subcore_barrier()
    pltpu.sync_copy(acc.at[pl.ds(row0, RPT)], out.at[cid, pl.ds(row0, RPT)])


def _post_body(p_ref, res_ref, g_ref, b_ref, o_ref, *, leaky):
    h = p_ref[0] + p_ref[1]
    if leaky:
        h = jnp.where(h > 0, h, LEAKY * h)
    mu = jnp.mean(h, axis=-1, keepdims=True)
    var = jnp.mean((h - mu) ** 2, axis=-1, keepdims=True)
    ln = (h - mu) * jax.lax.rsqrt(var + 1e-5) * g_ref[...] + b_ref[...]
    o_ref[...] = ln + res_ref[...]


def _post(p, res, gamma, beta, leaky):
    blk = 512
    return pl.pallas_call(
        functools.partial(_post_body, leaky=leaky),
        grid=(N_PAD // blk,),
        in_specs=[
            pl.BlockSpec((NC, blk, D), lambda i: (0, i, 0)),
            pl.BlockSpec((blk, D), lambda i: (i, 0)),
            pl.BlockSpec((D,), lambda i: (0,)),
            pl.BlockSpec((D,), lambda i: (0,)),
        ],
        out_specs=pl.BlockSpec((blk, D), lambda i: (i, 0)),
        out_shape=jax.ShapeDtypeStruct((N_PAD, D), jnp.float32),
    )(p, res, gamma, beta)


def _combine_body(p_ref, o_ref):
    o_ref[...] = p_ref[0] + p_ref[1]


def _combine(p):
    blk = 512
    return pl.pallas_call(
        _combine_body,
        grid=(N_PAD // blk,),
        in_specs=[pl.BlockSpec((NC, blk, D), lambda i: (0, i, 0))],
        out_specs=pl.BlockSpec((blk, D), lambda i: (i, 0)),
        out_shape=jax.ShapeDtypeStruct((N_PAD, D), jnp.float32),
    )(p)


def kernel(ego_embeddings, edge_index, edge_vals, ln0_gamma, ln0_beta, ln1_gamma, ln1_beta):
    pad = ((0, E_PAD - N_EDGES),)
    rows = jnp.pad(edge_index[0].astype(jnp.int32), pad).reshape(NW, NSUB, SUB)
    cols = jnp.pad(edge_index[1].astype(jnp.int32), pad).reshape(NW, NSUB, SUB)
    vals = jnp.pad(edge_vals, pad).reshape(NW, NSUB, SUB)
    e0 = jnp.pad(ego_embeddings, ((0, N_PAD - N_NODES), (0, 0)))
    p = _spmm(e0, rows, cols, vals)
    tmp = _combine(p)
    p = _spmm(tmp, cols, rows, vals)
    e1 = _post(p, e0, ln0_gamma, ln0_beta, leaky=True)
    p = _spmm(e1, rows, cols, vals)
    tmp = _combine(p)
    p = _spmm(tmp, cols, rows, vals)
    e2 = _post(p, e0, ln1_gamma, ln1_beta, leaky=False)
    return (e2[:N_USERS], e2[N_USERS:N_NODES])


# speedup vs baseline: 3.3765x; 1.1919x over previous
import functools

import jax
import jax.numpy as jnp
from jax import lax
from jax.experimental import pallas as pl
from jax.experimental.pallas import tpu as pltpu
from jax.experimental.pallas import tpu_sc as plsc

N_NODES = 10000
N_PAD = 10240
N_USERS = 5000
N_EDGES = 320000
D = 128
LEAKY = 0.2

NC = 2
NS = 16
NW = NC * NS
EPW = N_EDGES // NW
SUB = 128
EPW_PAD = 10240
NSUB = EPW_PAD // SUB
E_PAD = NW * EPW_PAD
GCH = 16
NG = NSUB // GCH
RPT = N_PAD // NS

_mesh = plsc.VectorSubcoreMesh(core_axis_name="c", subcore_axis_name="s")


@functools.partial(
    pl.kernel,
    out_type=jax.ShapeDtypeStruct((NC, N_PAD, D), jnp.float32),
    mesh=_mesh,
    scratch_types=[
        pltpu.VMEM((2, GCH, SUB), jnp.int32),
        pltpu.VMEM((2, GCH, SUB), jnp.int32),
        pltpu.VMEM((2, GCH, SUB), jnp.float32),
        pltpu.VMEM((2, SUB, D), jnp.float32),
        pltpu.VMEM_SHARED((N_PAD, D), jnp.float32),
        pltpu.SemaphoreType.DMA,
        pltpu.SemaphoreType.DMA,
        pltpu.SemaphoreType.DMA,
    ],
)
def _spmm(table, gidx, sidx, vals, out,
          gidx_v, sidx_v, vals_v, rows_v, acc, sem_g, sem_s, sem_i):
    cid = lax.axis_index("c")
    sid = lax.axis_index("s")
    wid = cid * NS + sid

    def _idx_load(g, buf):
        pltpu.async_copy(gidx.at[wid, pl.ds(g * GCH, GCH)], gidx_v.at[buf], sem_i)
        pltpu.async_copy(sidx.at[wid, pl.ds(g * GCH, GCH)], sidx_v.at[buf], sem_i)
        pltpu.async_copy(vals.at[wid, pl.ds(g * GCH, GCH)], vals_v.at[buf], sem_i)

    def _idx_wait(g, buf):
        pltpu.make_async_copy(gidx.at[wid, pl.ds(g * GCH, GCH)], gidx_v.at[buf], sem_i).wait()
        pltpu.make_async_copy(sidx.at[wid, pl.ds(g * GCH, GCH)], sidx_v.at[buf], sem_i).wait()
        pltpu.make_async_copy(vals.at[wid, pl.ds(g * GCH, GCH)], vals_v.at[buf], sem_i).wait()

    def _zrow(e, carry):
        for j in range(D // 16):
            rows_v[0, e, pl.ds(j * 16, 16)] = jnp.zeros((16,), jnp.float32)
        return carry

    lax.fori_loop(0, SUB, _zrow, 0)
    row0 = sid * RPT
    for k in range(RPT // SUB):
        pltpu.sync_copy(rows_v.at[0], acc.at[pl.ds(row0 + k * SUB, SUB)])
    plsc.subcore_barrier()

    _idx_load(0, 0)
    _idx_wait(0, 0)
    pltpu.async_copy(table.at[gidx_v.at[0, 0]], rows_v.at[0], sem_g)

    def _big(g, carry):
        cg = lax.rem(g, 2)
        for r in range(GCH):
            p = r % 2
            if r == 0:

                @pl.when(g < NG - 1)
                def _():
                    _idx_load(g + 1, 1 - cg)

            pltpu.make_async_copy(table.at[gidx_v.at[cg, r]], rows_v.at[p], sem_g).wait()
            if r == 0:

                @pl.when(g > 0)
                def _():
                    pltpu.make_async_copy(
                        rows_v.at[1 - p], acc.at[sidx_v.at[1 - cg, GCH - 1]], sem_s
                    ).wait()

            else:
                pltpu.make_async_copy(
                    rows_v.at[1 - p], acc.at[sidx_v.at[cg, r - 1]], sem_s
                ).wait()
            if r < GCH - 1:
                pltpu.async_copy(table.at[gidx_v.at[cg, r + 1]], rows_v.at[1 - p], sem_g)
            else:

                @pl.when(g < NG - 1)
                def _():
                    _idx_wait(g + 1, 1 - cg)
                    pltpu.async_copy(table.at[gidx_v.at[1 - cg, 0]], rows_v.at[1 - p], sem_g)

            def _scale(grp, c2):
                e0 = grp * 16
                val16 = vals_v[cg, r, pl.ds(e0, 16)]
                for l in range(16):
                    v = val16[l]
                    for j in range(D // 16):
                        rows_v[p, e0 + l, pl.ds(j * 16, 16)] = (
                            rows_v[p, e0 + l, pl.ds(j * 16, 16)] * v)
                return c2

            lax.fori_loop(0, SUB // 16, _scale, 0)
            pltpu.async_copy(rows_v.at[p], acc.at[sidx_v.at[cg, r]], sem_s, add=True)
        return carry

    lax.fori_loop(0, NG, _big, 0)
    pltpu.make_async_copy(rows_v.at[1], acc.at[sidx_v.at[lax.rem(NG - 1, 2), GCH - 1]], sem_s).wait()
    plsc.subcore_barrier()
    pltpu.sync_copy(acc.at[pl.ds(row0, RPT)], out.at[cid, pl.ds(row0, RPT)])


def _post_body(p_ref, res_ref, g_ref, b_ref, o_ref, *, leaky):
    h = p_ref[0] + p_ref[1]
    if leaky:
        h = jnp.where(h > 0, h, LEAKY * h)
    mu = jnp.mean(h, axis=-1, keepdims=True)
    var = jnp.mean((h - mu) ** 2, axis=-1, keepdims=True)
    ln = (h - mu) * jax.lax.rsqrt(var + 1e-5) * g_ref[...] + b_ref[...]
    o_ref[...] = ln + res_ref[...]


def _post(p, res, gamma, beta, leaky):
    blk = 512
    return pl.pallas_call(
        functools.partial(_post_body, leaky=leaky),
        grid=(N_PAD // blk,),
        in_specs=[
            pl.BlockSpec((NC, blk, D), lambda i: (0, i, 0)),
            pl.BlockSpec((blk, D), lambda i: (i, 0)),
            pl.BlockSpec((D,), lambda i: (0,)),
            pl.BlockSpec((D,), lambda i: (0,)),
        ],
        out_specs=pl.BlockSpec((blk, D), lambda i: (i, 0)),
        out_shape=jax.ShapeDtypeStruct((N_PAD, D), jnp.float32),
    )(p, res, gamma, beta)


def _combine_body(p_ref, o_ref):
    o_ref[...] = p_ref[0] + p_ref[1]


def _combine(p):
    blk = 512
    return pl.pallas_call(
        _combine_body,
        grid=(N_PAD // blk,),
        in_specs=[pl.BlockSpec((NC, blk, D), lambda i: (0, i, 0))],
        out_specs=pl.BlockSpec((blk, D), lambda i: (i, 0)),
        out_shape=jax.ShapeDtypeStruct((N_PAD, D), jnp.float32),
    )(p)


def kernel(ego_embeddings, edge_index, edge_vals, ln0_gamma, ln0_beta, ln1_gamma, ln1_beta):
    pad = ((0, E_PAD - N_EDGES),)
    rows = jnp.pad(edge_index[0].astype(jnp.int32), pad).reshape(NW, NSUB, SUB)
    cols = jnp.pad(edge_index[1].astype(jnp.int32), pad).reshape(NW, NSUB, SUB)
    vals = jnp.pad(edge_vals, pad).reshape(NW, NSUB, SUB)
    e0 = jnp.pad(ego_embeddings, ((0, N_PAD - N_NODES), (0, 0)))
    p = _spmm(e0, rows, cols, vals)
    tmp = _combine(p)
    p = _spmm(tmp, cols, rows, vals)
    e1 = _post(p, e0, ln0_gamma, ln0_beta, leaky=True)
    p = _spmm(e1, rows, cols, vals)
    tmp = _combine(p)
    p = _spmm(tmp, cols, rows, vals)
    e2 = _post(p, e0, ln1_gamma, ln1_beta, leaky=False)
    return (e2[:N_USERS], e2[N_USERS:N_NODES])
